# baseline (speedup 1.0000x reference)
import jax
import jax.numpy as jnp
from jax import lax
from jax.experimental import pallas as pl
from jax.experimental.pallas import tpu as pltpu
from jax.experimental.pallas import tpu_sc as plsc

_TAILLE = 16
_B, _S, _D = 16384, 50, 64
_NW = 32
_IB = _B // _NW
_FIRE = 10


def _sc_body(w_hbm, out_hbm, w_v, pat_v, sem):
    c = lax.axis_index("c")
    s = lax.axis_index("s")
    wid = s * 2 + c
    base = pl.multiple_of(wid * _IB, _IB)
    pltpu.sync_copy(w_hbm.at[pl.ds(base, _IB)], w_v)

    def build_k(k, carry):
        off = pl.multiple_of(k * 16, 16)
        wv = w_v[pl.ds(off, 16)]
        for j in range(_D):
            val = jnp.where((wv <= j) & (wv + _TAILLE > j),
                            jnp.float32(0.0), jnp.float32(1.0))
            pat_v[j, pl.ds(off, 16)] = val
        return carry

    lax.fori_loop(0, _IB // 16, build_k, 0)

    copies = []
    for s_i in range(_S):
        copies.append(
            pltpu.async_copy(pat_v, out_hbm.at[s_i, :, pl.ds(base, _IB)], sem))
        if len(copies) == _FIRE:
            for cp in copies:
                cp.wait()
            copies = []
    for cp in copies:
        cp.wait()


def kernel(ones_buf, w):
    del ones_buf
    mesh = plsc.VectorSubcoreMesh(core_axis_name="c", subcore_axis_name="s")
    sc_fill = pl.kernel(
        _sc_body,
        out_type=jax.ShapeDtypeStruct((_S, _D, _B), jnp.float32),
        mesh=mesh,
        scratch_types=[
            pltpu.VMEM((_IB,), jnp.int32),
            pltpu.VMEM((_D, _IB), jnp.float32),
            pltpu.SemaphoreType.DMA,
        ],
    )
    return jnp.transpose(sc_fill(w), (2, 0, 1))


# speedup vs baseline: 1.0031x; 1.0031x over previous
import jax
import jax.numpy as jnp
from jax import lax
from jax.experimental import pallas as pl
from jax.experimental.pallas import tpu as pltpu
from jax.experimental.pallas import tpu_sc as plsc

_TAILLE = 16
_B, _S, _D = 16384, 50, 64
_NW = 32
_IB = _B // _NW
_FIRE = 10


def _sc_body(w_hbm, out_hbm, w_v, pat_v, sem):
    c = lax.axis_index("c")
    s = lax.axis_index("s")
    wid = s * 2 + c
    base = pl.multiple_of(wid * _IB, _IB)
    pltpu.sync_copy(w_hbm.at[pl.ds(base, _IB)], w_v)

    def build_k(k, carry):
        off = pl.multiple_of(k * 16, 16)
        wv = w_v[pl.ds(off, 16)]
        for j in range(_D):
            val = jnp.where((wv <= j) & (wv + _TAILLE > j),
                            jnp.float32(0.0), jnp.float32(1.0))
            pat_v[j, pl.ds(off, 16)] = val
        return carry

    lax.fori_loop(0, _IB // 16, build_k, 0)

    copies = []
    for s_i in range(_S):
        copies.append(
            pltpu.async_copy(pat_v, out_hbm.at[s_i, :, pl.ds(base, _IB)], sem))
        if len(copies) > _FIRE:
            copies.pop(0).wait()
    for cp in copies:
        cp.wait()


def kernel(ones_buf, w):
    del ones_buf
    mesh = plsc.VectorSubcoreMesh(core_axis_name="c", subcore_axis_name="s")
    sc_fill = pl.kernel(
        _sc_body,
        out_type=jax.ShapeDtypeStruct((_S, _D, _B), jnp.float32),
        mesh=mesh,
        scratch_types=[
            pltpu.VMEM((_IB,), jnp.int32),
            pltpu.VMEM((_D, _IB), jnp.float32),
            pltpu.SemaphoreType.DMA,
        ],
    )
    return jnp.transpose(sc_fill(w), (2, 0, 1))
